# Initial kernel scaffold; baseline (speedup 1.0000x reference)
import jax
import jax.numpy as jnp
from jax.experimental import pallas as pl


def kernel(features, edge_index, W0, b0, W1, b1, W2, b2):
    raise NotImplementedError("write your pallas kernel here")


# speedup vs baseline: 4.2090x; 4.2090x over previous
import jax
import jax.numpy as jnp
from jax import lax
from jax.experimental import pallas as pl
from jax.experimental.pallas import tpu as pltpu
from jax.experimental.pallas import tpu_sc as plsc

N = 10000
E = 320000
NC = 2
NS = 16
NW = NC * NS
NPAD = 10240
RPT = NPAD // NS
ZR = RPT // 4
K = 80
EPT = E // NW
NCHUNK = EPT // K
BM = 400
F32 = jnp.float32

_MESH = plsc.VectorSubcoreMesh(core_axis_name="c", subcore_axis_name="s")


def _deg_body(src_hbm, dst_hbm, out_hbm, idx_v, ones_v, zrow_v, acc_out, acc_in):
    c = lax.axis_index("c")
    s = lax.axis_index("s")
    wid = s * NC + c

    @pl.loop(0, K, step=16)
    def _(i):
        ones_v[pl.ds(i, 16)] = jnp.ones((16,), F32)

    @pl.loop(0, RPT, step=16)
    def _(i):
        zrow_v[pl.ds(i, 16)] = jnp.zeros((16,), F32)

    r0 = s * RPT
    pltpu.sync_copy(zrow_v, acc_out.at[pl.ds(r0, RPT)])
    pltpu.sync_copy(zrow_v, acc_in.at[pl.ds(r0, RPT)])
    plsc.subcore_barrier()

    ebase = wid * EPT

    @pl.loop(0, NCHUNK)
    def _(j):
        b = ebase + j * K
        pltpu.sync_copy(src_hbm.at[pl.ds(b, K)], idx_v)
        pltpu.sync_copy(ones_v, acc_out.at[idx_v], add=True)
        pltpu.sync_copy(dst_hbm.at[pl.ds(b, K)], idx_v)
        pltpu.sync_copy(ones_v, acc_in.at[idx_v], add=True)

    plsc.subcore_barrier()
    pltpu.sync_copy(acc_out.at[pl.ds(r0, RPT)], out_hbm.at[c, 0, pl.ds(r0, RPT)])
    pltpu.sync_copy(acc_in.at[pl.ds(r0, RPT)], out_hbm.at[c, 1, pl.ds(r0, RPT)])


def _degrees(src, dst):
    fn = pl.kernel(
        _deg_body,
        out_type=jax.ShapeDtypeStruct((NC, 2, NPAD), F32),
        mesh=_MESH,
        scratch_types=[
            pltpu.VMEM((K,), jnp.int32),
            pltpu.VMEM((K,), F32),
            pltpu.VMEM((RPT,), F32),
            pltpu.VMEM_SHARED((NPAD,), F32),
            pltpu.VMEM_SHARED((NPAD,), F32),
        ],
    )
    return fn(src, dst)


def _make_mp_body(D):
    def body(h_hbm, src_hbm, dst_hbm, out_hbm, sidx, didx, rows, zbuf, acc):
        c = lax.axis_index("c")
        s = lax.axis_index("s")
        wid = s * NC + c

        @pl.loop(0, ZR)
        def _(r):
            @pl.loop(0, D, step=16)
            def _(c0):
                zbuf[r, pl.ds(c0, 16)] = jnp.zeros((16,), F32)

        r0 = s * RPT

        @pl.loop(0, RPT // ZR)
        def _(t):
            pltpu.sync_copy(zbuf, acc.at[pl.ds(r0 + t * ZR, ZR)])

        plsc.subcore_barrier()
        ebase = wid * EPT

        @pl.loop(0, NCHUNK)
        def _(j):
            b = ebase + j * K
            pltpu.sync_copy(src_hbm.at[pl.ds(b, K)], sidx)
            pltpu.sync_copy(dst_hbm.at[pl.ds(b, K)], didx)
            pltpu.sync_copy(h_hbm.at[sidx], rows)
            pltpu.sync_copy(rows, acc.at[didx], add=True)

        plsc.subcore_barrier()
        pltpu.sync_copy(acc.at[pl.ds(r0, RPT)], out_hbm.at[c, pl.ds(r0, RPT)])

    return body


def _message_pass(h, src, dst):
    D = h.shape[1]
    fn = pl.kernel(
        _make_mp_body(D),
        out_type=jax.ShapeDtypeStruct((NC, NPAD, D), F32),
        mesh=_MESH,
        scratch_types=[
            pltpu.VMEM((K,), jnp.int32),
            pltpu.VMEM((K,), jnp.int32),
            pltpu.VMEM((K, D), F32),
            pltpu.VMEM((ZR, D), F32),
            pltpu.VMEM_SHARED((NPAD, D), F32),
        ],
    )
    return fn(h, src, dst)


def _norms(degp):
    def body(d_ref, ns_ref, nd_ref):
        dsum = d_ref[0] + d_ref[1]
        ns_ref[...] = 1.0 / jnp.sqrt(jnp.maximum(dsum[0], 1.0))
        nd_ref[...] = 1.0 / jnp.sqrt(jnp.maximum(dsum[1], 1.0))

    blk = NPAD // 16
    return pl.pallas_call(
        body,
        grid=(16,),
        in_specs=[pl.BlockSpec((NC, 2, blk, 1), lambda i: (0, 0, i, 0))],
        out_specs=[
            pl.BlockSpec((blk, 1), lambda i: (i, 0)),
            pl.BlockSpec((blk, 1), lambda i: (i, 0)),
        ],
        out_shape=[
            jax.ShapeDtypeStruct((NPAD, 1), F32),
            jax.ShapeDtypeStruct((NPAD, 1), F32),
        ],
    )(degp)


def _mm_first(x, ns, w):
    def body(x_ref, ns_ref, w_ref, o_ref):
        o_ref[...] = lax.dot_general(
            x_ref[...] * ns_ref[...], w_ref[...],
            (((1,), (0,)), ((), ())), preferred_element_type=F32)

    return pl.pallas_call(
        body,
        grid=(N // BM,),
        in_specs=[
            pl.BlockSpec((BM, x.shape[1]), lambda i: (i, 0)),
            pl.BlockSpec((BM, 1), lambda i: (i, 0)),
            pl.BlockSpec(w.shape, lambda i: (0, 0)),
        ],
        out_specs=pl.BlockSpec((BM, w.shape[1]), lambda i: (i, 0)),
        out_shape=jax.ShapeDtypeStruct((N, w.shape[1]), F32),
    )(x, ns, w)


def _mm_mid(p, nd, b, ns, w):
    din = p.shape[2]

    def body(p_ref, nd_ref, b_ref, ns_ref, w_ref, o_ref):
        h = (p_ref[0] + p_ref[1]) * nd_ref[...] + b_ref[...]
        h = jnp.maximum(h, 0.0)
        o_ref[...] = lax.dot_general(
            h * ns_ref[...], w_ref[...],
            (((1,), (0,)), ((), ())), preferred_element_type=F32)

    return pl.pallas_call(
        body,
        grid=(N // BM,),
        in_specs=[
            pl.BlockSpec((NC, BM, din), lambda i: (0, i, 0)),
            pl.BlockSpec((BM, 1), lambda i: (i, 0)),
            pl.BlockSpec((1, din), lambda i: (0, 0)),
            pl.BlockSpec((BM, 1), lambda i: (i, 0)),
            pl.BlockSpec(w.shape, lambda i: (0, 0)),
        ],
        out_specs=pl.BlockSpec((BM, w.shape[1]), lambda i: (i, 0)),
        out_shape=jax.ShapeDtypeStruct((N, w.shape[1]), F32),
    )(p, nd, b, ns, w)


def _final(p, nd, b):
    dout = p.shape[2]

    def body(p_ref, nd_ref, b_ref, o_ref):
        o_ref[...] = (p_ref[0] + p_ref[1]) * nd_ref[...] + b_ref[...]

    return pl.pallas_call(
        body,
        grid=(N // BM,),
        in_specs=[
            pl.BlockSpec((NC, BM, dout), lambda i: (0, i, 0)),
            pl.BlockSpec((BM, 1), lambda i: (i, 0)),
            pl.BlockSpec((1, dout), lambda i: (0, 0)),
        ],
        out_specs=pl.BlockSpec((BM, dout), lambda i: (i, 0)),
        out_shape=jax.ShapeDtypeStruct((N, dout), F32),
    )(p, nd, b)


def kernel(features, edge_index, W0, b0, W1, b1, W2, b2):
    src = edge_index[0]
    dst = edge_index[1]

    degp = _degrees(src, dst)
    ns, nd = _norms(degp.reshape(NC, 2, NPAD, 1))

    w2p = jnp.pad(W2, ((0, 0), (0, 88)))
    b2p = jnp.pad(b2, (0, 88))

    h0 = _mm_first(features, ns, W0)
    p0 = _message_pass(h0, src, dst)
    h1 = _mm_mid(p0, nd, b0.reshape(1, -1), ns, W1)
    p1 = _message_pass(h1, src, dst)
    h2 = _mm_mid(p1, nd, b1.reshape(1, -1), ns, w2p)
    p2 = _message_pass(h2, src, dst)
    out = _final(p2, nd, b2p.reshape(1, -1))
    return out[:, :40]


# speedup vs baseline: 7.4286x; 1.7650x over previous
import jax
import jax.numpy as jnp
from jax import lax
from jax.experimental import pallas as pl
from jax.experimental.pallas import tpu as pltpu
from jax.experimental.pallas import tpu_sc as plsc

N = 10000
E = 320000
NC = 2
NS = 16
NW = NC * NS
NPAD = 10240
RPT = NPAD // NS
ZR = RPT // 4
K = 80
EPT = E // NW
NCHUNK = EPT // K
D = 128
BM = 400
F32 = jnp.float32

_MESH = plsc.VectorSubcoreMesh(core_axis_name="c", subcore_axis_name="s")


def _deg_body(src_hbm, dst_hbm, out_hbm, sidx0, sidx1, didx0, didx1,
              ones_v, zrow_v, acc_out, acc_in, sem0, sem1):
    c = lax.axis_index("c")
    s = lax.axis_index("s")
    wid = s * NC + c

    @pl.loop(0, K, step=16)
    def _(i):
        ones_v[pl.ds(i, 16)] = jnp.ones((16,), F32)

    @pl.loop(0, RPT, step=16)
    def _(i):
        zrow_v[pl.ds(i, 16)] = jnp.zeros((16,), F32)

    r0 = s * RPT
    pltpu.sync_copy(zrow_v, acc_out.at[pl.ds(r0, RPT)])
    pltpu.sync_copy(zrow_v, acc_in.at[pl.ds(r0, RPT)])
    plsc.subcore_barrier()

    ebase = wid * EPT

    def load(j, sbuf, dbuf):
        pltpu.sync_copy(src_hbm.at[pl.ds(ebase + j * K, K)], sbuf)
        pltpu.sync_copy(dst_hbm.at[pl.ds(ebase + j * K, K)], dbuf)

    def fire(sbuf, dbuf):
        pltpu.async_copy(ones_v, acc_out.at[sbuf], sem0, add=True)
        pltpu.async_copy(ones_v, acc_in.at[dbuf], sem1, add=True)

    def drain(sbuf, dbuf):
        pltpu.make_async_copy(ones_v, acc_out.at[sbuf], sem0).wait()
        pltpu.make_async_copy(ones_v, acc_in.at[dbuf], sem1).wait()

    load(0, sidx0, didx0)
    fire(sidx0, didx0)

    @pl.loop(0, NCHUNK // 2)
    def _(t):
        j = 2 * t
        load(j + 1, sidx1, didx1)
        fire(sidx1, didx1)
        drain(sidx0, didx0)
        load(j + 2, sidx0, didx0)
        fire(sidx0, didx0)
        drain(sidx1, didx1)

    drain(sidx0, didx0)

    plsc.subcore_barrier()
    pltpu.sync_copy(acc_out.at[pl.ds(r0, RPT)], out_hbm.at[c, 0, pl.ds(r0, RPT)])
    pltpu.sync_copy(acc_in.at[pl.ds(r0, RPT)], out_hbm.at[c, 1, pl.ds(r0, RPT)])


def _degrees(src, dst):
    fn = pl.kernel(
        _deg_body,
        out_type=jax.ShapeDtypeStruct((NC, 2, NPAD), F32),
        mesh=_MESH,
        scratch_types=[
            pltpu.VMEM((K,), jnp.int32),
            pltpu.VMEM((K,), jnp.int32),
            pltpu.VMEM((K,), jnp.int32),
            pltpu.VMEM((K,), jnp.int32),
            pltpu.VMEM((K,), F32),
            pltpu.VMEM((RPT,), F32),
            pltpu.VMEM_SHARED((NPAD,), F32),
            pltpu.VMEM_SHARED((NPAD,), F32),
            pltpu.SemaphoreType.DMA,
            pltpu.SemaphoreType.DMA,
        ],
    )
    return fn(src, dst)


def _mp_body(h_hbm, src_hbm, dst_hbm, out_hbm, sidx_all, didx0, didx1,
             rows0, rows1, acc, sem0, sem1):
    c = lax.axis_index("c")
    s = lax.axis_index("s")
    wid = s * NC + c

    @pl.loop(0, K)
    def _(r):
        @pl.loop(0, D, step=16)
        def _(c0):
            rows0[r, pl.ds(c0, 16)] = jnp.zeros((16,), F32)

    ebase = wid * EPT
    pltpu.sync_copy(src_hbm.at[pl.ds(ebase, EPT)], sidx_all)

    r0 = s * RPT

    @pl.loop(0, RPT // K)
    def _(t):
        pltpu.sync_copy(rows0, acc.at[pl.ds(r0 + t * K, K)])

    plsc.subcore_barrier()

    def gather(j, rbuf, sem):
        pltpu.async_copy(h_hbm.at[sidx_all.at[pl.ds(j * K, K)]], rbuf, sem)

    def gwait(j, rbuf, sem):
        pltpu.make_async_copy(h_hbm.at[sidx_all.at[pl.ds(j * K, K)]], rbuf, sem).wait()

    pltpu.sync_copy(dst_hbm.at[pl.ds(ebase, K)], didx0)
    gather(0, rows0, sem0)

    @pl.loop(0, NCHUNK // 2)
    def _(t):
        j = 2 * t
        pltpu.sync_copy(dst_hbm.at[pl.ds(ebase + (j + 1) * K, K)], didx1)
        gather(j + 1, rows1, sem1)
        gwait(j, rows0, sem0)
        pltpu.sync_copy(rows0, acc.at[didx0], add=True)
        pltpu.sync_copy(dst_hbm.at[pl.ds(ebase + (j + 2) * K, K)], didx0)
        gather(j + 2, rows0, sem0)
        gwait(j + 1, rows1, sem1)
        pltpu.sync_copy(rows1, acc.at[didx1], add=True)

    gwait(NCHUNK - 1, rows0, sem0)
    pltpu.sync_copy(rows0, acc.at[didx0], add=True)

    plsc.subcore_barrier()
    pltpu.sync_copy(acc.at[pl.ds(r0, RPT)], out_hbm.at[c, pl.ds(r0, RPT)])


def _message_pass(h, src, dst):
    fn = pl.kernel(
        _mp_body,
        out_type=jax.ShapeDtypeStruct((NC, NPAD, D), F32),
        mesh=_MESH,
        scratch_types=[
            pltpu.VMEM((EPT,), jnp.int32),
            pltpu.VMEM((K,), jnp.int32),
            pltpu.VMEM((K,), jnp.int32),
            pltpu.VMEM((K, D), F32),
            pltpu.VMEM((K, D), F32),
            pltpu.VMEM_SHARED((NPAD, D), F32),
            pltpu.SemaphoreType.DMA,
            pltpu.SemaphoreType.DMA,
        ],
    )
    return fn(h, src, dst)


def _norms(degp):
    def body(d_ref, ns_ref, nd_ref):
        dsum = d_ref[0] + d_ref[1]
        ns_ref[...] = 1.0 / jnp.sqrt(jnp.maximum(dsum[0], 1.0))
        nd_ref[...] = 1.0 / jnp.sqrt(jnp.maximum(dsum[1], 1.0))

    blk = NPAD // 16
    return pl.pallas_call(
        body,
        grid=(16,),
        in_specs=[pl.BlockSpec((NC, 2, blk, 1), lambda i: (0, 0, i, 0))],
        out_specs=[
            pl.BlockSpec((blk, 1), lambda i: (i, 0)),
            pl.BlockSpec((blk, 1), lambda i: (i, 0)),
        ],
        out_shape=[
            jax.ShapeDtypeStruct((NPAD, 1), F32),
            jax.ShapeDtypeStruct((NPAD, 1), F32),
        ],
    )(degp)


def _mm_plain(x, w):
    def body(x_ref, w_ref, o_ref):
        o_ref[...] = lax.dot_general(
            x_ref[...], w_ref[...],
            (((1,), (0,)), ((), ())), preferred_element_type=F32)

    return pl.pallas_call(
        body,
        grid=(N // BM,),
        in_specs=[
            pl.BlockSpec((BM, x.shape[1]), lambda i: (i, 0)),
            pl.BlockSpec(w.shape, lambda i: (0, 0)),
        ],
        out_specs=pl.BlockSpec((BM, w.shape[1]), lambda i: (i, 0)),
        out_shape=jax.ShapeDtypeStruct((N, w.shape[1]), F32),
    )(x, w)


def _scale(u, ns):
    def body(u_ref, ns_ref, o_ref):
        o_ref[...] = u_ref[...] * ns_ref[...]

    return pl.pallas_call(
        body,
        grid=(N // BM,),
        in_specs=[
            pl.BlockSpec((BM, u.shape[1]), lambda i: (i, 0)),
            pl.BlockSpec((BM, 1), lambda i: (i, 0)),
        ],
        out_specs=pl.BlockSpec((BM, u.shape[1]), lambda i: (i, 0)),
        out_shape=jax.ShapeDtypeStruct(u.shape, F32),
    )(u, ns)


def _mm_mid(p, nd, b, ns, w):
    din = p.shape[2]

    def body(p_ref, nd_ref, b_ref, ns_ref, w_ref, o_ref):
        h = (p_ref[0] + p_ref[1]) * nd_ref[...] + b_ref[...]
        h = jnp.maximum(h, 0.0)
        o_ref[...] = lax.dot_general(
            h * ns_ref[...], w_ref[...],
            (((1,), (0,)), ((), ())), preferred_element_type=F32)

    return pl.pallas_call(
        body,
        grid=(N // BM,),
        in_specs=[
            pl.BlockSpec((NC, BM, din), lambda i: (0, i, 0)),
            pl.BlockSpec((BM, 1), lambda i: (i, 0)),
            pl.BlockSpec((1, din), lambda i: (0, 0)),
            pl.BlockSpec((BM, 1), lambda i: (i, 0)),
            pl.BlockSpec(w.shape, lambda i: (0, 0)),
        ],
        out_specs=pl.BlockSpec((BM, w.shape[1]), lambda i: (i, 0)),
        out_shape=jax.ShapeDtypeStruct((N, w.shape[1]), F32),
    )(p, nd, b, ns, w)


def _final(p, nd, b):
    dout = p.shape[2]

    def body(p_ref, nd_ref, b_ref, o_ref):
        o_ref[...] = (p_ref[0] + p_ref[1]) * nd_ref[...] + b_ref[...]

    return pl.pallas_call(
        body,
        grid=(N // BM,),
        in_specs=[
            pl.BlockSpec((NC, BM, dout), lambda i: (0, i, 0)),
            pl.BlockSpec((BM, 1), lambda i: (i, 0)),
            pl.BlockSpec((1, dout), lambda i: (0, 0)),
        ],
        out_specs=pl.BlockSpec((BM, dout), lambda i: (i, 0)),
        out_shape=jax.ShapeDtypeStruct((N, dout), F32),
    )(p, nd, b)


def kernel(features, edge_index, W0, b0, W1, b1, W2, b2):
    src = edge_index[0]
    dst = edge_index[1]

    w2p = jnp.pad(W2, ((0, 0), (0, 88)))
    b2p = jnp.pad(b2, (0, 88))

    u0 = _mm_plain(features, W0)
    degp = _degrees(src, dst)
    ns, nd = _norms(degp.reshape(NC, 2, NPAD, 1))

    h0 = _scale(u0, ns)
    p0 = _message_pass(h0, src, dst)
    h1 = _mm_mid(p0, nd, b0.reshape(1, -1), ns, W1)
    p1 = _message_pass(h1, src, dst)
    h2 = _mm_mid(p1, nd, b1.reshape(1, -1), ns, w2p)
    p2 = _message_pass(h2, src, dst)
    out = _final(p2, nd, b2p.reshape(1, -1))
    return out[:, :40]
